# baseline (device time: 8648 ns/iter reference)
import jax
import jax.numpy as jnp
from jax import lax
from jax.experimental import pallas as pl
from jax.experimental.pallas import tpu as pltpu

N_DEV = 16
KTAPS = 4
HALO = KTAPS - 1
TAIL = 8
NB = 4
BS = 128


def kernel(x, k):
    b, s, c = x.shape

    def body(x_ref, k_ref, out_blk, halo_ref, send_buf, send_sem, recv_sem,
             credit_sem):
        my_i = lax.axis_index("i")
        left = (my_i - 1) % N_DEV
        right = (my_i + 1) % N_DEV
        is_first = my_i == 0
        is_last = my_i == N_DEV - 1

        i = pl.program_id(0)
        j = NB - 1 - i
        barrier_sem = pltpu.get_barrier_semaphore()

        rdma = pltpu.make_async_remote_copy(
            src_ref=send_buf,
            dst_ref=halo_ref,
            send_sem=send_sem,
            recv_sem=recv_sem,
            device_id=(right,),
            device_id_type=pl.DeviceIdType.MESH,
        )

        @pl.when(jnp.logical_and(i == 0, jnp.logical_not(is_first)))
        def _():
            pl.semaphore_signal(
                barrier_sem, inc=1,
                device_id=(left,), device_id_type=pl.DeviceIdType.MESH,
            )

        @pl.when(i == 0)
        def _():
            send_buf[...] = x_ref[:, s - TAIL:, :]

        @pl.when(jnp.logical_and(i == 0, jnp.logical_not(is_last)))
        def _():
            pl.semaphore_wait(barrier_sem, 1)
            rdma.start()

        @pl.when(jnp.logical_and(i == 0, is_first))
        def _():
            halo_ref[...] = jnp.zeros((b, TAIL, c), halo_ref.dtype)

        kv = k_ref[...].astype(jnp.bfloat16)

        @pl.when(i < NB - 1)
        def _():
            base = pl.multiple_of(j * BS, BS)
            prev = pl.multiple_of(j * BS - TAIL, TAIL)
            blk = x_ref[:, pl.ds(base, BS), :].astype(jnp.bfloat16)
            carry = x_ref[:, pl.ds(prev, TAIL), :].astype(jnp.bfloat16)
            pad = jnp.concatenate([carry[:, TAIL - HALO:, :], blk], axis=1)
            acc = pad[:, HALO:, :] * kv[KTAPS - 1, :]
            for t in range(KTAPS - 1):
                acc = acc + pad[:, t:t + BS, :] * kv[t, :]
            out_blk[...] = acc / (1.0 + jnp.exp(-acc)).astype(jnp.bfloat16)

        @pl.when(i == NB - 1)
        def _():
            x_head = x_ref[:, :BS, :].astype(jnp.bfloat16)
            acc = x_head[:, HALO:, :] * kv[KTAPS - 1, :]
            for t in range(KTAPS - 1):
                acc = acc + x_head[:, t:t + BS - HALO, :] * kv[t, :]
            out_blk[:, HALO:, :] = (
                acc / (1.0 + jnp.exp(-acc)).astype(jnp.bfloat16)
            )

            @pl.when(jnp.logical_not(is_first))
            def _():
                rdma.wait_recv()
                pl.semaphore_signal(
                    credit_sem, inc=1,
                    device_id=(left,), device_id_type=pl.DeviceIdType.MESH,
                )

            halo = halo_ref[:, TAIL - HALO:, :].astype(jnp.bfloat16)
            pad = jnp.concatenate([halo, x_head[:, :HALO, :]], axis=1)
            acc0 = pad[:, HALO:, :] * kv[KTAPS - 1, :]
            for t in range(KTAPS - 1):
                acc0 = acc0 + pad[:, t:t + HALO, :] * kv[t, :]
            out_blk[:, :HALO, :] = (
                acc0 / (1.0 + jnp.exp(-acc0)).astype(jnp.bfloat16)
            )

            @pl.when(jnp.logical_not(is_last))
            def _():
                rdma.wait_send()
                pl.semaphore_wait(credit_sem, 1)

    return pl.pallas_call(
        body,
        grid=(NB,),
        out_shape=jax.ShapeDtypeStruct((b, s, c), jnp.bfloat16),
        in_specs=[
            pl.BlockSpec(memory_space=pltpu.MemorySpace.VMEM),
            pl.BlockSpec(memory_space=pltpu.MemorySpace.VMEM),
        ],
        out_specs=pl.BlockSpec((b, BS, c), lambda i: (0, NB - 1 - i, 0)),
        scratch_shapes=[
            pltpu.VMEM((b, TAIL, c), x.dtype),
            pltpu.VMEM((b, TAIL, c), x.dtype),
            pltpu.SemaphoreType.DMA,
            pltpu.SemaphoreType.DMA,
            pltpu.SemaphoreType.REGULAR,
        ],
        compiler_params=pltpu.CompilerParams(collective_id=0),
    )(x, k)


# device time: 7197 ns/iter; 1.2016x vs baseline; 1.2016x over previous
import functools

import jax
import jax.numpy as jnp
from jax import lax
from jax.experimental import pallas as pl
from jax.experimental.pallas import tpu as pltpu

N_DEV = 16
KTAPS = 4
HALO = KTAPS - 1
TAIL = 8
HEAD = 16


def kernel(x, k):
    b, s, c = x.shape

    def body(x_hbm, k_hbm, out_hbm, x_vmem, k_vmem, out_vmem, halo_ref,
             send_buf, msa_blocker, local_sems, send_sem, recv_sem):
        msa_blocker[0, :] = jnp.zeros((1024,), jnp.float32)
        my_i = lax.axis_index("i")
        left = (my_i - 1) % N_DEV
        right = (my_i + 1) % N_DEV
        is_first = my_i == 0
        is_last = my_i == N_DEV - 1

        barrier_sem = pltpu.get_barrier_semaphore()

        @pl.when(jnp.logical_not(is_first))
        def _():
            pl.semaphore_signal(
                barrier_sem, inc=1,
                device_id=(left,), device_id_type=pl.DeviceIdType.MESH,
            )

        tail_cp = pltpu.make_async_copy(
            x_hbm.at[:, pl.ds(s - TAIL, TAIL), :], send_buf, local_sems.at[0]
        )
        tail_cp.start()
        x_cp = pltpu.make_async_copy(x_hbm, x_vmem, local_sems.at[1])
        x_cp.start()
        k_cp = pltpu.make_async_copy(k_hbm, k_vmem, local_sems.at[2])
        k_cp.start()

        rdma = pltpu.make_async_remote_copy(
            src_ref=send_buf,
            dst_ref=halo_ref,
            send_sem=send_sem,
            recv_sem=recv_sem,
            device_id=(right,),
            device_id_type=pl.DeviceIdType.MESH,
        )

        tail_cp.wait()

        @pl.when(jnp.logical_not(is_last))
        def _():
            pl.semaphore_wait(barrier_sem, 1)
            rdma.start()

        @pl.when(is_first)
        def _():
            halo_ref[...] = jnp.zeros((b, TAIL, c), halo_ref.dtype)

        x_cp.wait()
        k_cp.wait()

        x_val = x_vmem[...].astype(jnp.bfloat16)
        kv = k_vmem[...].astype(jnp.bfloat16)
        acc = x_val[:, HALO:, :] * kv[KTAPS - 1, :]
        for t in range(KTAPS - 1):
            acc = acc + x_val[:, t:t + s - HALO, :] * kv[t, :]
        out_vmem[:, HALO:, :] = acc / (1.0 + jnp.exp(-acc)).astype(jnp.bfloat16)

        out_cp_main = pltpu.make_async_copy(
            out_vmem.at[:, pl.ds(HEAD, s - HEAD), :],
            out_hbm.at[:, pl.ds(HEAD, s - HEAD), :],
            local_sems.at[3],
        )
        out_cp_main.start()

        @functools.partial(
            pl.run_scoped, credit_sem=pltpu.SemaphoreType.REGULAR
        )
        def _(credit_sem):
            @pl.when(jnp.logical_not(is_first))
            def _():
                rdma.wait_recv()
                pl.semaphore_signal(
                    credit_sem, inc=1,
                    device_id=(left,), device_id_type=pl.DeviceIdType.MESH,
                )

            halo = halo_ref[:, TAIL - HALO:, :].astype(jnp.bfloat16)
            pad = jnp.concatenate([halo, x_val[:, :HALO, :]], axis=1)
            acc0 = pad[:, HALO:, :] * kv[KTAPS - 1, :]
            for t in range(KTAPS - 1):
                acc0 = acc0 + pad[:, t:t + HALO, :] * kv[t, :]
            out_vmem[:, :HALO, :] = (
                acc0 / (1.0 + jnp.exp(-acc0)).astype(jnp.bfloat16)
            )

            out_cp_head = pltpu.make_async_copy(
                out_vmem.at[:, pl.ds(0, HEAD), :],
                out_hbm.at[:, pl.ds(0, HEAD), :],
                local_sems.at[4],
            )
            out_cp_head.start()
            out_cp_head.wait()
            out_cp_main.wait()

            @pl.when(jnp.logical_not(is_last))
            def _():
                rdma.wait_send()
                pl.semaphore_wait(credit_sem, 1)

    return pl.pallas_call(
        body,
        out_shape=jax.ShapeDtypeStruct((b, s, c), jnp.bfloat16),
        in_specs=[
            pl.BlockSpec(memory_space=pltpu.MemorySpace.HBM),
            pl.BlockSpec(memory_space=pltpu.MemorySpace.HBM),
        ],
        out_specs=pl.BlockSpec(memory_space=pltpu.MemorySpace.HBM),
        scratch_shapes=[
            pltpu.VMEM((b, s, c), x.dtype),
            pltpu.VMEM(k.shape, k.dtype),
            pltpu.VMEM((b, s, c), jnp.bfloat16),
            pltpu.VMEM((b, TAIL, c), x.dtype),
            pltpu.VMEM((b, TAIL, c), x.dtype),
            pltpu.VMEM((15 * 1024, 1024), jnp.float32),
            pltpu.SemaphoreType.DMA((5,)),
            pltpu.SemaphoreType.DMA,
            pltpu.SemaphoreType.DMA,
        ],
        compiler_params=pltpu.CompilerParams(collective_id=0),
    )(x, k)


# device time: 7042 ns/iter; 1.2281x vs baseline; 1.0220x over previous
import functools

import jax
import jax.numpy as jnp
from jax import lax
from jax.experimental import pallas as pl
from jax.experimental.pallas import tpu as pltpu

N_DEV = 16
KTAPS = 4
HALO = KTAPS - 1
TAIL = 8


def kernel(x, k):
    b, s, c = x.shape

    def body(x_ref, k_ref, out_ref, halo_ref, send_buf, send_sem, recv_sem):
        my_i = lax.axis_index("i")
        left = (my_i - 1) % N_DEV
        right = (my_i + 1) % N_DEV
        is_first = my_i == 0
        is_last = my_i == N_DEV - 1

        barrier_sem = pltpu.get_barrier_semaphore()

        @pl.when(jnp.logical_not(is_first))
        def _():
            pl.semaphore_signal(
                barrier_sem, inc=1,
                device_id=(left,), device_id_type=pl.DeviceIdType.MESH,
            )

        send_buf[...] = x_ref[:, s - TAIL:, :]
        rdma = pltpu.make_async_remote_copy(
            src_ref=send_buf,
            dst_ref=halo_ref,
            send_sem=send_sem,
            recv_sem=recv_sem,
            device_id=(right,),
            device_id_type=pl.DeviceIdType.MESH,
        )

        @pl.when(jnp.logical_not(is_last))
        def _():
            pl.semaphore_wait(barrier_sem, 1)
            rdma.start()

        @pl.when(is_first)
        def _():
            halo_ref[...] = jnp.zeros((b, TAIL, c), halo_ref.dtype)

        x_val = x_ref[...].astype(jnp.bfloat16)
        kv = k_ref[...].astype(jnp.bfloat16)
        acc = x_val[:, HALO:, :] * kv[KTAPS - 1, :]
        for t in range(KTAPS - 1):
            acc = acc + x_val[:, t:t + s - HALO, :] * kv[t, :]
        out_ref[:, HALO:, :] = acc / (1.0 + jnp.exp(-acc)).astype(jnp.bfloat16)

        @functools.partial(
            pl.run_scoped, credit_sem=pltpu.SemaphoreType.REGULAR
        )
        def _(credit_sem):
            @pl.when(jnp.logical_not(is_first))
            def _():
                rdma.wait_recv()
                pl.semaphore_signal(
                    credit_sem, inc=1,
                    device_id=(left,), device_id_type=pl.DeviceIdType.MESH,
                )

            halo = halo_ref[:, TAIL - HALO:, :].astype(jnp.bfloat16)
            pad = jnp.concatenate([halo, x_val[:, :HALO, :]], axis=1)
            acc0 = pad[:, HALO:, :] * kv[KTAPS - 1, :]
            for t in range(KTAPS - 1):
                acc0 = acc0 + pad[:, t:t + HALO, :] * kv[t, :]
            out_ref[:, :HALO, :] = (
                acc0 / (1.0 + jnp.exp(-acc0)).astype(jnp.bfloat16)
            )

            @pl.when(jnp.logical_not(is_last))
            def _():
                rdma.wait_send()
                pl.semaphore_wait(credit_sem, 1)

    return pl.pallas_call(
        body,
        out_shape=jax.ShapeDtypeStruct((b, s, c), jnp.bfloat16),
        in_specs=[
            pl.BlockSpec(memory_space=pltpu.MemorySpace.VMEM),
            pl.BlockSpec(memory_space=pltpu.MemorySpace.VMEM),
        ],
        out_specs=pl.BlockSpec(memory_space=pltpu.MemorySpace.VMEM),
        scratch_shapes=[
            pltpu.VMEM((b, TAIL, c), x.dtype),
            pltpu.VMEM((b, TAIL, c), x.dtype),
            pltpu.SemaphoreType.DMA,
            pltpu.SemaphoreType.DMA,
        ],
        compiler_params=pltpu.CompilerParams(collective_id=0),
    )(x, k)
